# baseline (speedup 1.0000x reference)
import functools

import jax
import jax.numpy as jnp
from jax import lax
from jax.experimental import pallas as pl
from jax.experimental.pallas import tpu as pltpu
from jax.experimental.pallas import tpu_sc as plsc

_LEVELS = 7
_ROWS, _COLS = 32768, 768


_BLK_ROWS = 2048
_CHUNK = 64


def _zq_body(b_ref, z_ref, zq_ref):
    b0 = b_ref[0]
    step = (b_ref[_LEVELS - 1] - b_ref[0]) * (1.0 / (_LEVELS - 1))
    scale = 1.0 / step
    shift = -b0 * scale - 0.5
    for r in range(0, _BLK_ROWS, _CHUNK):
        z = z_ref[r:r + _CHUNK, :]
        idx_f = jnp.clip(jnp.ceil(z * scale + shift), 0.0, float(_LEVELS - 1))
        zq_ref[r:r + _CHUNK, :] = idx_f * step + b0


def _tc_zq(z, boundaries):
    grid = (_ROWS // _BLK_ROWS,)
    return pl.pallas_call(
        _zq_body,
        grid=grid,
        in_specs=[
            pl.BlockSpec(memory_space=pltpu.SMEM),
            pl.BlockSpec((_BLK_ROWS, _COLS), lambda i: (i, 0)),
        ],
        out_specs=pl.BlockSpec((_BLK_ROWS, _COLS), lambda i: (i, 0)),
        out_shape=jax.ShapeDtypeStruct((_ROWS, _COLS), z.dtype),
    )(boundaries, z)


_TOT = _ROWS * _COLS
_NW = 32
_PER_W = _TOT // _NW
_SC_C = 16384
_NCH = _PER_W // _SC_C
_ROUNDS = _NCH // 2
_KOFF = 8


def _sc_idx_body(z_hbm, sc_hbm, sh_hbm, out_hbm, zbuf, obuf, svec, hvec,
                 sem_in, sem_out):
    wid = lax.axis_index("s") * 2 + lax.axis_index("c")
    base = wid * _PER_W
    pltpu.sync_copy(sc_hbm, svec)
    pltpu.sync_copy(sh_hbm, hvec)
    scale = svec[...]
    shift = hvec[...]

    def compute_chunk(b):
        def vec_step(i, _):
            zv = zbuf[b, pl.ds(i * 16, 16)]
            iv = (zv * scale + shift).astype(jnp.int32)
            iv = jnp.minimum(jnp.maximum(iv, _KOFF) - _KOFF, _LEVELS - 1)
            obuf[b, pl.ds(i * 16, 16)] = iv
            return 0
        lax.fori_loop(0, _SC_C // 16, vec_step, 0)

    for b in range(2):
        pltpu.async_copy(z_hbm.at[pl.ds(base + b * _SC_C, _SC_C)],
                         zbuf.at[b], sem_in)

    def round_step(g2, _):
        for b in range(2):
            g = g2 * 2 + b
            off = base + g * _SC_C
            pltpu.make_async_copy(z_hbm.at[pl.ds(0, _SC_C)], zbuf.at[b],
                                  sem_in).wait()

            @pl.when(g2 > 0)
            def _wait_out():
                pltpu.make_async_copy(obuf.at[b],
                                      out_hbm.at[pl.ds(0, _SC_C)],
                                      sem_out).wait()

            compute_chunk(b)
            pltpu.async_copy(obuf.at[b], out_hbm.at[pl.ds(off, _SC_C)],
                             sem_out)

            @pl.when(g2 < _ROUNDS - 1)
            def _next_in():
                pltpu.async_copy(
                    z_hbm.at[pl.ds(off + 2 * _SC_C, _SC_C)],
                    zbuf.at[b], sem_in)
        return 0

    lax.fori_loop(0, _ROUNDS, round_step, 0)
    for b in range(2):
        pltpu.make_async_copy(obuf.at[b], out_hbm.at[pl.ds(0, _SC_C)],
                              sem_out).wait()


def _sc_idx(z_flat, scale16, shift16):
    mesh = plsc.VectorSubcoreMesh(core_axis_name="c", subcore_axis_name="s")
    call = functools.partial(
        pl.kernel,
        out_type=jax.ShapeDtypeStruct((_TOT,), jnp.int32),
        mesh=mesh,
        scratch_types=[
            pltpu.VMEM((2, _SC_C), jnp.float32),
            pltpu.VMEM((2, _SC_C), jnp.int32),
            pltpu.VMEM((16,), jnp.float32),
            pltpu.VMEM((16,), jnp.float32),
            pltpu.SemaphoreType.DMA,
            pltpu.SemaphoreType.DMA,
        ],
    )(_sc_idx_body)
    return call(z_flat, scale16, shift16)


def kernel(z, boundaries):
    b0 = boundaries[0]
    step = (boundaries[_LEVELS - 1] - b0) * (1.0 / (_LEVELS - 1))
    scale = 1.0 / step
    shift = -b0 * scale + 0.5 + _KOFF
    scale16 = jnp.full((16,), scale, jnp.float32)
    shift16 = jnp.full((16,), shift, jnp.float32)

    zq = _tc_zq(z, boundaries)
    idx = _sc_idx(z.reshape(_TOT), scale16, shift16)
    return zq, idx.reshape(_ROWS, _COLS)


# speedup vs baseline: 1.2965x; 1.2965x over previous
import functools

import jax
import jax.numpy as jnp
from jax import lax
from jax.experimental import pallas as pl
from jax.experimental.pallas import tpu as pltpu
from jax.experimental.pallas import tpu_sc as plsc

_LEVELS = 7
_ROWS, _COLS = 32768, 768


_BLK_ROWS = 2048
_CHUNK = 64


def _zq_body(b_ref, z_ref, zq_ref):
    b0 = b_ref[0]
    step = (b_ref[_LEVELS - 1] - b_ref[0]) * (1.0 / (_LEVELS - 1))
    scale = 1.0 / step
    shift = -b0 * scale - 0.5
    for r in range(0, _BLK_ROWS, _CHUNK):
        z = z_ref[r:r + _CHUNK, :]
        idx_f = jnp.clip(jnp.ceil(z * scale + shift), 0.0, float(_LEVELS - 1))
        zq_ref[r:r + _CHUNK, :] = idx_f * step + b0


def _tc_zq(z, boundaries):
    grid = (_ROWS // _BLK_ROWS,)
    return pl.pallas_call(
        _zq_body,
        grid=grid,
        in_specs=[
            pl.BlockSpec(memory_space=pltpu.SMEM),
            pl.BlockSpec((_BLK_ROWS, _COLS), lambda i: (i, 0)),
        ],
        out_specs=pl.BlockSpec((_BLK_ROWS, _COLS), lambda i: (i, 0)),
        out_shape=jax.ShapeDtypeStruct((_ROWS, _COLS), z.dtype),
    )(boundaries, z)


_TOT = _ROWS * _COLS
_NW = 32
_PER_W = _TOT // _NW
_SC_C = 16384
_NCH = _PER_W // _SC_C
_ROUNDS = _NCH // 2
_KOFF = 8


def _sc_idx_body(z_hbm, sc_hbm, sh_hbm, out_hbm, zbuf, obuf, svec, hvec,
                 sem_in, sem_out):
    wid = lax.axis_index("s") * 2 + lax.axis_index("c")
    base = wid * _PER_W
    pltpu.sync_copy(sc_hbm, svec)
    pltpu.sync_copy(sh_hbm, hvec)
    scale = svec[...]
    shift = hvec[...]

    def compute_chunk(b):
        @plsc.parallel_loop(0, _SC_C // 16, 1, unroll=8)
        def _vec_step(i):
            zv = zbuf[b, pl.ds(i * 16, 16)]
            iv = (zv * scale + shift).astype(jnp.int32)
            iv = jnp.minimum(jnp.maximum(iv, _KOFF) - _KOFF, _LEVELS - 1)
            obuf[b, pl.ds(i * 16, 16)] = iv

    for b in range(2):
        pltpu.async_copy(z_hbm.at[pl.ds(base + b * _SC_C, _SC_C)],
                         zbuf.at[b], sem_in)

    def round_step(g2, _):
        for b in range(2):
            g = g2 * 2 + b
            off = base + g * _SC_C
            pltpu.make_async_copy(z_hbm.at[pl.ds(0, _SC_C)], zbuf.at[b],
                                  sem_in).wait()

            @pl.when(g2 > 0)
            def _wait_out():
                pltpu.make_async_copy(obuf.at[b],
                                      out_hbm.at[pl.ds(0, _SC_C)],
                                      sem_out).wait()

            compute_chunk(b)
            pltpu.async_copy(obuf.at[b], out_hbm.at[pl.ds(off, _SC_C)],
                             sem_out)

            @pl.when(g2 < _ROUNDS - 1)
            def _next_in():
                pltpu.async_copy(
                    z_hbm.at[pl.ds(off + 2 * _SC_C, _SC_C)],
                    zbuf.at[b], sem_in)
        return 0

    lax.fori_loop(0, _ROUNDS, round_step, 0)
    for b in range(2):
        pltpu.make_async_copy(obuf.at[b], out_hbm.at[pl.ds(0, _SC_C)],
                              sem_out).wait()


def _sc_idx(z_flat, scale16, shift16):
    mesh = plsc.VectorSubcoreMesh(core_axis_name="c", subcore_axis_name="s")
    call = functools.partial(
        pl.kernel,
        out_type=jax.ShapeDtypeStruct((_TOT,), jnp.int32),
        mesh=mesh,
        scratch_types=[
            pltpu.VMEM((2, _SC_C), jnp.float32),
            pltpu.VMEM((2, _SC_C), jnp.int32),
            pltpu.VMEM((16,), jnp.float32),
            pltpu.VMEM((16,), jnp.float32),
            pltpu.SemaphoreType.DMA,
            pltpu.SemaphoreType.DMA,
        ],
    )(_sc_idx_body)
    return call(z_flat, scale16, shift16)


def kernel(z, boundaries):
    b0 = boundaries[0]
    step = (boundaries[_LEVELS - 1] - b0) * (1.0 / (_LEVELS - 1))
    scale = 1.0 / step
    shift = -b0 * scale + 0.5 + _KOFF
    scale16 = jnp.full((16,), scale, jnp.float32)
    shift16 = jnp.full((16,), shift, jnp.float32)

    zq = _tc_zq(z, boundaries)
    idx = _sc_idx(z.reshape(_TOT), scale16, shift16)
    return zq, idx.reshape(_ROWS, _COLS)


# speedup vs baseline: 3.1543x; 2.4330x over previous
import functools

import jax
import jax.numpy as jnp
from jax import lax
from jax.experimental import pallas as pl
from jax.experimental.pallas import tpu as pltpu
from jax.experimental.pallas import tpu_sc as plsc

_LEVELS = 7
_ROWS, _COLS = 32768, 768


_BLK_ROWS = 2048
_CHUNK = 64


def _zq_body(b_ref, z_ref, zq_ref):
    b0 = b_ref[0]
    step = (b_ref[_LEVELS - 1] - b_ref[0]) * (1.0 / (_LEVELS - 1))
    scale = 1.0 / step
    shift = -b0 * scale - 0.5
    for r in range(0, _BLK_ROWS, _CHUNK):
        z = z_ref[r:r + _CHUNK, :]
        idx_f = jnp.clip(jnp.ceil(z * scale + shift), 0.0, float(_LEVELS - 1))
        zq_ref[r:r + _CHUNK, :] = idx_f * step + b0


def _tc_zq(z, boundaries):
    grid = (_ROWS // _BLK_ROWS,)
    return pl.pallas_call(
        _zq_body,
        grid=grid,
        in_specs=[
            pl.BlockSpec(memory_space=pltpu.SMEM),
            pl.BlockSpec((_BLK_ROWS, _COLS), lambda i: (i, 0)),
        ],
        out_specs=pl.BlockSpec((_BLK_ROWS, _COLS), lambda i: (i, 0)),
        out_shape=jax.ShapeDtypeStruct((_ROWS, _COLS), z.dtype),
    )(boundaries, z)


_NW = 32
_W_ROWS = _ROWS // _NW
_RCH = 32
_NCH = _W_ROWS // _RCH
_ROUNDS = _NCH // 2
_VECS = _COLS // 16


def _sc_idx_body(z_hbm, sc_hbm, sh_hbm, out_hbm, zbuf, obuf, svec, hvec,
                 sem_in, sem_out):
    wid = lax.axis_index("s") * 2 + lax.axis_index("c")
    base = wid * _W_ROWS
    pltpu.sync_copy(sc_hbm, svec)
    pltpu.sync_copy(sh_hbm, hvec)
    scale = svec[...]
    shift = hvec[...]
    lo = jnp.full((16,), 0.5, jnp.float32)
    hi = jnp.full((16,), _LEVELS - 0.25, jnp.float32)

    def compute_chunk(b):
        def row_step(rr, _):
            @plsc.parallel_loop(0, _VECS, 1, unroll=8)
            def _vec_step(i):
                zv = zbuf[b, rr, pl.ds(i * 16, 16)]
                t = jnp.minimum(jnp.maximum(zv * scale + shift, lo), hi)
                obuf[b, rr, pl.ds(i * 16, 16)] = t.astype(jnp.int32)
            return 0
        lax.fori_loop(0, _RCH, row_step, 0)

    for b in range(2):
        pltpu.async_copy(z_hbm.at[pl.ds(base + b * _RCH, _RCH)],
                         zbuf.at[b], sem_in)

    def round_step(g2, _):
        for b in range(2):
            g = g2 * 2 + b
            row0 = base + g * _RCH
            pltpu.make_async_copy(z_hbm.at[pl.ds(0, _RCH)], zbuf.at[b],
                                  sem_in).wait()

            @pl.when(g2 > 0)
            def _wait_out():
                pltpu.make_async_copy(obuf.at[b],
                                      out_hbm.at[pl.ds(0, _RCH)],
                                      sem_out).wait()

            compute_chunk(b)
            pltpu.async_copy(obuf.at[b], out_hbm.at[pl.ds(row0, _RCH)],
                             sem_out)

            @pl.when(g2 < _ROUNDS - 1)
            def _next_in():
                pltpu.async_copy(
                    z_hbm.at[pl.ds(row0 + 2 * _RCH, _RCH)],
                    zbuf.at[b], sem_in)
        return 0

    lax.fori_loop(0, _ROUNDS, round_step, 0)
    for b in range(2):
        pltpu.make_async_copy(obuf.at[b], out_hbm.at[pl.ds(0, _RCH)],
                              sem_out).wait()


def _sc_idx(z, scale16, shift16):
    mesh = plsc.VectorSubcoreMesh(core_axis_name="c", subcore_axis_name="s")
    call = functools.partial(
        pl.kernel,
        out_type=jax.ShapeDtypeStruct((_ROWS, _COLS), jnp.int32),
        mesh=mesh,
        scratch_types=[
            pltpu.VMEM((2, _RCH, _COLS), jnp.float32),
            pltpu.VMEM((2, _RCH, _COLS), jnp.int32),
            pltpu.VMEM((16,), jnp.float32),
            pltpu.VMEM((16,), jnp.float32),
            pltpu.SemaphoreType.DMA,
            pltpu.SemaphoreType.DMA,
        ],
    )(_sc_idx_body)
    return call(z, scale16, shift16)


def kernel(z, boundaries):
    b0 = boundaries[0]
    step = (boundaries[_LEVELS - 1] - b0) * (1.0 / (_LEVELS - 1))
    scale = 1.0 / step
    shift = -b0 * scale + 0.5
    scale16 = jnp.full((16,), scale, jnp.float32)
    shift16 = jnp.full((16,), shift, jnp.float32)

    zq = _tc_zq(z, boundaries)
    idx = _sc_idx(z, scale16, shift16)
    return zq, idx


# speedup vs baseline: 3.1873x; 1.0104x over previous
import functools

import jax
import jax.numpy as jnp
from jax import lax
from jax.experimental import pallas as pl
from jax.experimental.pallas import tpu as pltpu
from jax.experimental.pallas import tpu_sc as plsc

_LEVELS = 7
_ROWS, _COLS = 32768, 768


_BLK_ROWS = 2048
_CHUNK = 64


def _zq_body(b_ref, z_ref, zq_ref):
    b0 = b_ref[0]
    step = (b_ref[_LEVELS - 1] - b_ref[0]) * (1.0 / (_LEVELS - 1))
    scale = 1.0 / step
    shift = -b0 * scale - 0.5
    for r in range(0, _BLK_ROWS, _CHUNK):
        z = z_ref[r:r + _CHUNK, :]
        idx_f = jnp.clip(jnp.ceil(z * scale + shift), 0.0, float(_LEVELS - 1))
        zq_ref[r:r + _CHUNK, :] = idx_f * step + b0


def _tc_zq(z, boundaries):
    grid = (_ROWS // _BLK_ROWS,)
    return pl.pallas_call(
        _zq_body,
        grid=grid,
        in_specs=[
            pl.BlockSpec(memory_space=pltpu.SMEM),
            pl.BlockSpec((_BLK_ROWS, _COLS), lambda i: (i, 0)),
        ],
        out_specs=pl.BlockSpec((_BLK_ROWS, _COLS), lambda i: (i, 0)),
        out_shape=jax.ShapeDtypeStruct((_ROWS, _COLS), z.dtype),
    )(boundaries, z)


_NW = 32
_W_ROWS = _ROWS // _NW
_RCH = 16
_NBUF = 4
_NCH = _W_ROWS // _RCH
_ROUNDS = _NCH // _NBUF
_VECS = _COLS // 16


def _sc_idx_body(z_hbm, sc_hbm, sh_hbm, out_hbm, zbuf, obuf, svec, hvec,
                 sem_in, sem_out):
    wid = lax.axis_index("s") * 2 + lax.axis_index("c")
    base = wid * _W_ROWS
    pltpu.sync_copy(sc_hbm, svec)
    pltpu.sync_copy(sh_hbm, hvec)
    scale = svec[...]
    shift = hvec[...]
    lo = jnp.full((16,), 0.5, jnp.float32)
    hi = jnp.full((16,), _LEVELS - 0.25, jnp.float32)

    def compute_chunk(b):
        def row_step(rr, _):
            @plsc.parallel_loop(0, _VECS, 1, unroll=8)
            def _vec_step(i):
                zv = zbuf[b, rr, pl.ds(i * 16, 16)]
                t = jnp.minimum(jnp.maximum(zv * scale + shift, lo), hi)
                obuf[b, rr, pl.ds(i * 16, 16)] = t.astype(jnp.int32)
            return 0
        lax.fori_loop(0, _RCH, row_step, 0)

    for b in range(_NBUF):
        pltpu.async_copy(z_hbm.at[pl.ds(base + b * _RCH, _RCH)],
                         zbuf.at[b], sem_in)

    def round_step(gr, _):
        for b in range(_NBUF):
            g = gr * _NBUF + b
            row0 = base + g * _RCH
            pltpu.make_async_copy(z_hbm.at[pl.ds(0, _RCH)], zbuf.at[b],
                                  sem_in).wait()

            @pl.when(gr > 0)
            def _wait_out():
                pltpu.make_async_copy(obuf.at[b],
                                      out_hbm.at[pl.ds(0, _RCH)],
                                      sem_out).wait()

            compute_chunk(b)
            pltpu.async_copy(obuf.at[b], out_hbm.at[pl.ds(row0, _RCH)],
                             sem_out)

            @pl.when(gr < _ROUNDS - 1)
            def _next_in():
                pltpu.async_copy(
                    z_hbm.at[pl.ds(row0 + _NBUF * _RCH, _RCH)],
                    zbuf.at[b], sem_in)
        return 0

    lax.fori_loop(0, _ROUNDS, round_step, 0)
    for b in range(_NBUF):
        pltpu.make_async_copy(obuf.at[b], out_hbm.at[pl.ds(0, _RCH)],
                              sem_out).wait()


def _sc_idx(z, scale16, shift16):
    mesh = plsc.VectorSubcoreMesh(core_axis_name="c", subcore_axis_name="s")
    call = functools.partial(
        pl.kernel,
        out_type=jax.ShapeDtypeStruct((_ROWS, _COLS), jnp.int32),
        mesh=mesh,
        scratch_types=[
            pltpu.VMEM((_NBUF, _RCH, _COLS), jnp.float32),
            pltpu.VMEM((_NBUF, _RCH, _COLS), jnp.int32),
            pltpu.VMEM((16,), jnp.float32),
            pltpu.VMEM((16,), jnp.float32),
            pltpu.SemaphoreType.DMA,
            pltpu.SemaphoreType.DMA,
        ],
    )(_sc_idx_body)
    return call(z, scale16, shift16)


def kernel(z, boundaries):
    b0 = boundaries[0]
    step = (boundaries[_LEVELS - 1] - b0) * (1.0 / (_LEVELS - 1))
    scale = 1.0 / step
    shift = -b0 * scale + 0.5
    scale16 = jnp.full((16,), scale, jnp.float32)
    shift16 = jnp.full((16,), shift, jnp.float32)

    zq = _tc_zq(z, boundaries)
    idx = _sc_idx(z, scale16, shift16)
    return zq, idx


# speedup vs baseline: 3.5133x; 1.1023x over previous
import functools

import jax
import jax.numpy as jnp
from jax import lax
from jax.experimental import pallas as pl
from jax.experimental.pallas import tpu as pltpu
from jax.experimental.pallas import tpu_sc as plsc

_LEVELS = 7
_ROWS, _COLS = 32768, 768

_NW = 32
_W_ROWS = _ROWS // _NW
_RCH = 8
_NBUF = 4
_NCH = _W_ROWS // _RCH
_ROUNDS = _NCH // _NBUF
_VECS = _COLS // 16


def _sc_body(z_hbm, sc_hbm, sh_hbm, st_hbm, b0_hbm, idx_hbm, zq_hbm,
             zbuf, ibuf, qbuf, svec, hvec, tvec, bvec, sem_in, sem_out):
    wid = lax.axis_index("s") * 2 + lax.axis_index("c")
    base = wid * _W_ROWS
    pltpu.sync_copy(sc_hbm, svec)
    pltpu.sync_copy(sh_hbm, hvec)
    pltpu.sync_copy(st_hbm, tvec)
    pltpu.sync_copy(b0_hbm, bvec)
    scale = svec[...]
    shift = hvec[...]
    stepv = tvec[...]
    b0v = bvec[...]
    lo = jnp.full((16,), 0.5, jnp.float32)
    hi = jnp.full((16,), _LEVELS - 0.25, jnp.float32)

    def compute_chunk(b):
        def row_step(rr, _):
            @plsc.parallel_loop(0, _VECS, 1, unroll=8)
            def _vec_step(i):
                zv = zbuf[b, rr, pl.ds(i * 16, 16)]
                t = jnp.minimum(jnp.maximum(zv * scale + shift, lo), hi)
                iv = t.astype(jnp.int32)
                ibuf[b, rr, pl.ds(i * 16, 16)] = iv
                qbuf[b, rr, pl.ds(i * 16, 16)] = (
                    iv.astype(jnp.float32) * stepv + b0v)
            return 0
        lax.fori_loop(0, _RCH, row_step, 0)

    for b in range(_NBUF):
        pltpu.async_copy(z_hbm.at[pl.ds(base + b * _RCH, _RCH)],
                         zbuf.at[b], sem_in)

    def round_step(gr, _):
        for b in range(_NBUF):
            g = gr * _NBUF + b
            row0 = base + g * _RCH
            pltpu.make_async_copy(z_hbm.at[pl.ds(0, _RCH)], zbuf.at[b],
                                  sem_in).wait()

            @pl.when(gr > 0)
            def _wait_out():
                pltpu.make_async_copy(ibuf.at[b],
                                      idx_hbm.at[pl.ds(0, _RCH)],
                                      sem_out).wait()
                pltpu.make_async_copy(qbuf.at[b],
                                      zq_hbm.at[pl.ds(0, _RCH)],
                                      sem_out).wait()

            compute_chunk(b)
            pltpu.async_copy(ibuf.at[b], idx_hbm.at[pl.ds(row0, _RCH)],
                             sem_out)
            pltpu.async_copy(qbuf.at[b], zq_hbm.at[pl.ds(row0, _RCH)],
                             sem_out)

            @pl.when(gr < _ROUNDS - 1)
            def _next_in():
                pltpu.async_copy(
                    z_hbm.at[pl.ds(row0 + _NBUF * _RCH, _RCH)],
                    zbuf.at[b], sem_in)
        return 0

    lax.fori_loop(0, _ROUNDS, round_step, 0)
    for b in range(_NBUF):
        pltpu.make_async_copy(ibuf.at[b], idx_hbm.at[pl.ds(0, _RCH)],
                              sem_out).wait()
        pltpu.make_async_copy(qbuf.at[b], zq_hbm.at[pl.ds(0, _RCH)],
                              sem_out).wait()


def _sc_quantize(z, scale16, shift16, step16, b016):
    mesh = plsc.VectorSubcoreMesh(core_axis_name="c", subcore_axis_name="s")
    call = functools.partial(
        pl.kernel,
        out_type=[
            jax.ShapeDtypeStruct((_ROWS, _COLS), jnp.int32),
            jax.ShapeDtypeStruct((_ROWS, _COLS), jnp.float32),
        ],
        mesh=mesh,
        scratch_types=[
            pltpu.VMEM((_NBUF, _RCH, _COLS), jnp.float32),
            pltpu.VMEM((_NBUF, _RCH, _COLS), jnp.int32),
            pltpu.VMEM((_NBUF, _RCH, _COLS), jnp.float32),
            pltpu.VMEM((16,), jnp.float32),
            pltpu.VMEM((16,), jnp.float32),
            pltpu.VMEM((16,), jnp.float32),
            pltpu.VMEM((16,), jnp.float32),
            pltpu.SemaphoreType.DMA,
            pltpu.SemaphoreType.DMA,
        ],
    )(_sc_body)
    return call(z, scale16, shift16, step16, b016)


def kernel(z, boundaries):
    b0 = boundaries[0]
    step = (boundaries[_LEVELS - 1] - b0) * (1.0 / (_LEVELS - 1))
    scale = 1.0 / step
    shift = -b0 * scale + 0.5
    scale16 = jnp.full((16,), scale, jnp.float32)
    shift16 = jnp.full((16,), shift, jnp.float32)
    step16 = jnp.full((16,), step, jnp.float32)
    b016 = jnp.full((16,), b0, jnp.float32)

    idx, zq = _sc_quantize(z, scale16, shift16, step16, b016)
    return zq, idx
